# baseline (speedup 1.0000x reference)
import jax
import jax.numpy as jnp
import math
from jax.experimental import pallas as pl


def _edge_conv(x, edge_index, e, valid, W, b, n):
    src = edge_index[0]
    dst = edge_index[1]
    msg = jnp.concatenate([x[src], x[dst], e], axis=1)
    edge_out = (msg @ W + b) * valid[:, None]
    deg = jnp.clip(jax.ops.segment_sum(valid, dst, num_segments=n), 1.0)
    node_out = jax.ops.segment_sum(edge_out, dst, num_segments=n) / deg[:, None]
    return node_out, edge_out


def _gat_score(x, edge_index, valid, W, a, n):
    h = (x @ W)[:, 0]
    src = edge_index[0]
    dst = edge_index[1]
    alpha = jax.nn.leaky_relu(h[src] * a[0] + h[dst] * a[1], 0.2)
    alpha = jnp.where(valid > 0, alpha, -1e9)
    m = jax.ops.segment_max(alpha, dst, num_segments=n)
    m = jnp.where(jnp.isfinite(m), m, 0.0)
    ex = jnp.exp(alpha - m[dst]) * valid
    den = jnp.clip(jax.ops.segment_sum(ex, dst, num_segments=n), 1e-16)
    num = jax.ops.segment_sum(ex * h[src], dst, num_segments=n)
    return num / den


def _sag_pool(x, edge_index, edge_attr, valid, batch, W, a):
    n = x.shape[0]
    k = int(math.ceil(0.5 * n))
    score = _gat_score(x, edge_index, valid, W, a, n)
    _, perm = jax.lax.top_k(score, k)
    x_pool = x[perm] * jnp.tanh(score[perm])[:, None]
    mask = jnp.zeros(n, dtype=bool).at[perm].set(True)
    new_idx = jnp.zeros(n, edge_index.dtype).at[perm].set(jnp.arange(k, dtype=edge_index.dtype))
    src = edge_index[0]
    dst = edge_index[1]
    emask = (mask[src] & mask[dst]).astype(x.dtype) * valid
    ei_new = jnp.stack([new_idx[src], new_idx[dst]])
    return x_pool, ei_new, edge_attr, emask, batch[perm]


def _gmp(x, batch):
    return jax.ops.segment_max(x, batch, num_segments=1)


def _gap(x, batch):
    s = jax.ops.segment_sum(x, batch, num_segments=1)
    c = jnp.clip(jax.ops.segment_sum(jnp.ones((x.shape[0], 1)), batch, num_segments=1), 1.0)
    return s / c


def _mlp_kernel(l_ref, w1_ref, b1_ref, w2_ref, b2_ref, w3_ref, b3_ref, o_ref):
    h = jnp.maximum(l_ref[...] @ w1_ref[...] + b1_ref[...], 0.0)
    h = jnp.maximum(h @ w2_ref[...] + b2_ref[...], 0.0)
    o_ref[...] = h @ w3_ref[...] + b3_ref[...]


def kernel(node_feat, node_level, edge_index, edge_feat, W1, b1, W2, b2, W3, b3, W4, b4, gat2_W, gat2_a, gat3_W, gat3_a, gat4_W, gat4_a, Wl1, bl1, Wl2, bl2, Wl3, bl3):
    n = node_feat.shape[0]
    x = jnp.concatenate([node_feat.reshape(n, -1), node_level], axis=1)
    valid = jnp.ones(edge_index.shape[1], jnp.float32)
    x1, e1 = _edge_conv(x, edge_index, edge_feat, valid, W1, b1, n)
    x1 = jax.nn.relu(x1); e1 = jax.nn.relu(e1)
    x2, e2 = _edge_conv(x1, edge_index, e1, valid, W2, b2, n)
    x2 = jax.nn.relu(x2); e2 = jax.nn.relu(e2)
    batch = jnp.zeros(n, jnp.int32)
    x2p, ei2, e2p, v2, bt2 = _sag_pool(x2, edge_index, e2, valid, batch, gat2_W, gat2_a)
    l1 = jnp.concatenate([_gmp(x2p, bt2), _gap(x2p, bt2)], axis=1)
    x3, e3 = _edge_conv(x2p, ei2, e2p, v2, W3, b3, x2p.shape[0])
    x3 = jax.nn.relu(x3); e3 = jax.nn.relu(e3)
    x3p, ei3, e3p, v3, bt3 = _sag_pool(x3, ei2, e3, v2, bt2, gat3_W, gat3_a)
    l2 = jnp.concatenate([_gmp(x3p, bt3), _gap(x3p, bt3)], axis=1)
    x4, e4 = _edge_conv(x3p, ei3, e3p, v3, W4, b4, x3p.shape[0])
    x4 = jax.nn.relu(x4); e4 = jax.nn.relu(e4)
    x4p, ei4, e4p, v4, bt4 = _sag_pool(x4, ei3, e4, v3, bt3, gat4_W, gat4_a)
    l3 = jnp.concatenate([_gmp(x4p, bt4), _gap(x4p, bt4)], axis=1)
    l = l1 + l2 + l3
    out = pl.pallas_call(
        _mlp_kernel,
        out_shape=jax.ShapeDtypeStruct((1, Wl3.shape[1]), jnp.float32),
    )(l, Wl1, bl1.reshape(1, -1), Wl2, bl2.reshape(1, -1), Wl3, bl3.reshape(1, -1))
    return out


# speedup vs baseline: 5.6727x; 5.6727x over previous
import functools
import math

import jax
import jax.numpy as jnp
from jax import lax
from jax.experimental import pallas as pl
from jax.experimental.pallas import tpu as pltpu
from jax.experimental.pallas import tpu_sc as plsc

N = 10000
E = 320000
H = 128
NP = 10240
DUMP = 10000
NW = 32
CHUNK = 128
CPW = 80
EPW = CPW * CHUNK
EP = NW * EPW
NEG = -1e30
NROW = NP // 16

_mesh = plsc.VectorSubcoreMesh(core_axis_name="c", subcore_axis_name="s")


def _sc_gather128(table, idx2d):
    @functools.partial(
        pl.kernel, mesh=_mesh,
        out_type=jax.ShapeDtypeStruct((EP, 128), jnp.float32),
        scratch_types=[pltpu.VMEM((CPW, CHUNK), jnp.int32),
                       pltpu.VMEM((4, CHUNK, 128), jnp.float32),
                       pltpu.SemaphoreType.DMA,
                       pltpu.SemaphoreType.DMA],
    )
    def k(table_hbm, idx_hbm, out_hbm, idx_v, rbuf, sem_g, sem_o):
        wid = lax.axis_index("s") * 2 + lax.axis_index("c")
        pltpu.sync_copy(idx_hbm.at[pl.ds(wid * CPW, CPW)], idx_v)
        base = wid * EPW
        for g0 in range(0, CPW, 4):
            gh = [pltpu.async_copy(table_hbm.at[idx_v.at[g0 + t]],
                                   rbuf.at[t], sem_g) for t in range(4)]
            oh = []
            for t in range(4):
                gh[t].wait()
                oh.append(pltpu.async_copy(
                    rbuf.at[t],
                    out_hbm.at[pl.ds(base + (g0 + t) * CHUNK, CHUNK)], sem_o))
            for h in oh:
                h.wait()

    return k(table, idx2d)


def _sc_scatter128(rows, idx2d, zeros):
    @functools.partial(
        pl.kernel, mesh=_mesh,
        out_type=jax.ShapeDtypeStruct((2, NP, 128), jnp.float32),
        scratch_types=[pltpu.VMEM((CPW, CHUNK), jnp.int32),
                       pltpu.VMEM((2, CHUNK, 128), jnp.float32),
                       pltpu.VMEM_SHARED((NP, 128), jnp.float32),
                       pltpu.SemaphoreType.DMA],
    )
    def k(rows_hbm, idx_hbm, zeros_hbm, out_hbm, idx_v, rbuf, shared, sem):
        cid = lax.axis_index("c")
        sid = lax.axis_index("s")
        wid = sid * 2 + cid

        @pl.when(sid == 0)
        def _():
            pltpu.sync_copy(zeros_hbm, shared)

        pltpu.sync_copy(idx_hbm.at[pl.ds(wid * CPW, CPW)], idx_v)
        plsc.subcore_barrier()
        base = wid * EPW
        h_cur = pltpu.async_copy(rows_hbm.at[pl.ds(base, CHUNK)], rbuf.at[0], sem)
        for j in range(CPW):
            h_next = None
            if j + 1 < CPW:
                h_next = pltpu.async_copy(
                    rows_hbm.at[pl.ds(base + (j + 1) * CHUNK, CHUNK)],
                    rbuf.at[(j + 1) % 2], sem)
            h_cur.wait()
            pltpu.sync_copy(rbuf.at[j % 2], shared.at[idx_v.at[j]], add=True)
            h_cur = h_next
        plsc.subcore_barrier()
        pltpu.sync_copy(shared.at[pl.ds(sid * NROW, NROW)],
                        out_hbm.at[cid].at[pl.ds(sid * NROW, NROW)])

    return k(rows, idx2d, zeros)


def _sc_deg(idx2d, ones_row, zeros128):
    @functools.partial(
        pl.kernel, mesh=_mesh,
        out_type=jax.ShapeDtypeStruct((2, NP, 128), jnp.float32),
        scratch_types=[pltpu.VMEM((CPW, CHUNK), jnp.int32),
                       pltpu.VMEM((CHUNK, 128), jnp.float32),
                       pltpu.VMEM_SHARED((NP, 128), jnp.float32)],
    )
    def k(idx_hbm, ones_hbm, zeros_hbm, out_hbm, idx_v, ones_v, shared):
        cid = lax.axis_index("c")
        sid = lax.axis_index("s")
        wid = sid * 2 + cid

        @pl.when(sid == 0)
        def _():
            pltpu.sync_copy(zeros_hbm, shared)

        pltpu.sync_copy(idx_hbm.at[pl.ds(wid * CPW, CPW)], idx_v)
        pltpu.sync_copy(ones_hbm, ones_v)
        plsc.subcore_barrier()
        for j in range(CPW):
            pltpu.sync_copy(ones_v, shared.at[idx_v.at[j]], add=True)
        plsc.subcore_barrier()
        pltpu.sync_copy(shared.at[pl.ds(sid * NROW, NROW)],
                        out_hbm.at[cid].at[pl.ds(sid * NROW, NROW)])

    return k(idx2d, ones_row, zeros128)


def _sc_gat_gather(h, m, src1d, dst1d):
    @functools.partial(
        pl.kernel, mesh=_mesh,
        compiler_params=pltpu.CompilerParams(needs_layout_passes=False),
        out_type=[jax.ShapeDtypeStruct((EP,), jnp.float32),
                  jax.ShapeDtypeStruct((EP,), jnp.float32),
                  jax.ShapeDtypeStruct((EP,), jnp.float32)],
        scratch_types=[pltpu.VMEM((NP,), jnp.float32),
                       pltpu.VMEM((NP,), jnp.float32),
                       pltpu.VMEM((EPW,), jnp.int32),
                       pltpu.VMEM((EPW,), jnp.int32),
                       pltpu.VMEM((EPW,), jnp.float32),
                       pltpu.VMEM((EPW,), jnp.float32),
                       pltpu.VMEM((EPW,), jnp.float32)],
    )
    def k(h_hbm, m_hbm, src_hbm, dst_hbm, hs_hbm, hd_hbm, vv_hbm,
          h_v, m_v, src_v, dst_v, hs_v, hd_v, vv_v):
        wid = lax.axis_index("s") * 2 + lax.axis_index("c")
        base = wid * EPW
        pltpu.sync_copy(h_hbm, h_v)
        pltpu.sync_copy(m_hbm, m_v)
        pltpu.sync_copy(src_hbm.at[pl.ds(base, EPW)], src_v)
        pltpu.sync_copy(dst_hbm.at[pl.ds(base, EPW)], dst_v)

        def body(g, carry):
            o = g * 16
            is_ = src_v[pl.ds(o, 16)]
            id_ = dst_v[pl.ds(o, 16)]
            hs = plsc.load_gather(h_v, [is_])
            hd = plsc.load_gather(h_v, [id_])
            ms = plsc.load_gather(m_v, [is_])
            md = plsc.load_gather(m_v, [id_])
            hs_v[pl.ds(o, 16)] = hs
            hd_v[pl.ds(o, 16)] = hd
            vv_v[pl.ds(o, 16)] = ms * md
            return carry

        lax.fori_loop(0, EPW // 16, body, 0)
        pltpu.sync_copy(hs_v, hs_hbm.at[pl.ds(base, EPW)])
        pltpu.sync_copy(hd_v, hd_hbm.at[pl.ds(base, EPW)])
        pltpu.sync_copy(vv_v, vv_hbm.at[pl.ds(base, EPW)])

    return k(h, m, src1d, dst1d)


def _tc_prep(x, m, Ws, Wd, b):
    Fin = x.shape[1]
    BLK = 1024
    masked = m is not None

    def body(*refs):
        if masked:
            x_ref, m_ref, ws_ref, wd_ref, b_ref, xs_ref, xd_ref = refs
        else:
            x_ref, ws_ref, wd_ref, b_ref, xs_ref, xd_ref = refs
        xb = x_ref[...]
        xs = jnp.dot(xb, ws_ref[...], preferred_element_type=jnp.float32)
        xd = jnp.dot(xb, wd_ref[...], preferred_element_type=jnp.float32) + b_ref[...]
        if masked:
            keep = m_ref[...] > 0
            xs = jnp.where(keep, xs, NEG)
            xd = jnp.where(keep, xd, NEG)
        xs_ref[...] = xs
        xd_ref[...] = xd

    in_specs = [pl.BlockSpec((BLK, Fin), lambda i: (i, 0))]
    args = [x]
    if masked:
        in_specs.append(pl.BlockSpec((BLK, 1), lambda i: (i, 0)))
        args.append(m)
    in_specs += [pl.BlockSpec((Fin, H), lambda i: (0, 0)),
                 pl.BlockSpec((Fin, H), lambda i: (0, 0)),
                 pl.BlockSpec((1, H), lambda i: (0, 0))]
    args += [Ws, Wd, b.reshape(1, H)]
    return pl.pallas_call(
        body,
        grid=(NP // BLK,),
        in_specs=in_specs,
        out_specs=[pl.BlockSpec((BLK, H), lambda i: (i, 0)),
                   pl.BlockSpec((BLK, H), lambda i: (i, 0))],
        out_shape=[jax.ShapeDtypeStruct((NP, H), jnp.float32),
                   jax.ShapeDtypeStruct((NP, H), jnp.float32)],
    )(*args)


def _tc_edge(gs, gd, linprev, dst2d, We, relu_prev):
    BLK = 1024
    Dp = linprev.shape[1]
    with_gd = gd is not None

    def body(*refs):
        if with_gd:
            gs_ref, gd_ref, lp_ref, dst_ref, we_ref, lin_ref, dstp_ref = refs
        else:
            gs_ref, lp_ref, dst_ref, we_ref, lin_ref, dstp_ref = refs
        ep = lp_ref[...]
        if relu_prev:
            ep = jnp.maximum(ep, 0.0)
        mm = jnp.dot(ep, we_ref[...], preferred_element_type=jnp.float32)
        gsb = gs_ref[...]
        ok = gsb[:, 0:1] > -1e29
        if with_gd:
            gdb = gd_ref[...]
            ok = jnp.logical_and(ok, gdb[:, 0:1] > -1e29)
            lin_ref[...] = gsb + gdb + mm
        else:
            lin_ref[...] = gsb + mm
        okr = jnp.reshape(ok, (BLK // 128, 128))
        dstp_ref[...] = jnp.where(okr, dst_ref[...], DUMP)

    in_specs = [pl.BlockSpec((BLK, 128), lambda i: (i, 0))]
    args = [gs]
    if with_gd:
        in_specs.append(pl.BlockSpec((BLK, 128), lambda i: (i, 0)))
        args.append(gd)
    in_specs += [pl.BlockSpec((BLK, Dp), lambda i: (i, 0)),
                 pl.BlockSpec((BLK // 128, 128), lambda i: (i, 0)),
                 pl.BlockSpec((Dp, H), lambda i: (0, 0))]
    args += [linprev, dst2d, We]
    return pl.pallas_call(
        body,
        grid=(EP // BLK,),
        in_specs=in_specs,
        out_specs=[pl.BlockSpec((BLK, 128), lambda i: (i, 0)),
                   pl.BlockSpec((BLK // 128, 128), lambda i: (i, 0))],
        out_shape=[jax.ShapeDtypeStruct((EP, 128), jnp.float32),
                   jax.ShapeDtypeStruct((EP // 128, 128), jnp.int32)],
    )(*args)


def _tc_fin(acc, deg, xdLb, gatW):
    BLK = 1024
    with_xd = xdLb is not None

    def body(*refs):
        if with_xd:
            a_ref, d_ref, xd_ref, gw_ref, x_ref, h_ref = refs
        else:
            a_ref, d_ref, gw_ref, x_ref, h_ref = refs
        acc_b = a_ref[0] + a_ref[1]
        deg_b = jnp.maximum(d_ref[0][:, 0:1] + d_ref[1][:, 0:1], 1.0)
        xb = acc_b / deg_b
        if with_xd:
            xb = xb + xd_ref[...]
        xb = jnp.maximum(xb, 0.0)
        x_ref[...] = xb
        h_ref[...] = jnp.dot(xb, gw_ref[...], preferred_element_type=jnp.float32)

    in_specs = [pl.BlockSpec((2, BLK, H), lambda i: (0, i, 0)),
                pl.BlockSpec((2, BLK, H), lambda i: (0, i, 0))]
    args = [acc, deg]
    if with_xd:
        in_specs.append(pl.BlockSpec((BLK, H), lambda i: (i, 0)))
        args.append(xdLb)
    in_specs.append(pl.BlockSpec((H, 1), lambda i: (0, 0)))
    args.append(gatW)
    return pl.pallas_call(
        body,
        grid=(NP // BLK,),
        in_specs=in_specs,
        out_specs=[pl.BlockSpec((BLK, H), lambda i: (i, 0)),
                   pl.BlockSpec((BLK, 1), lambda i: (i, 0))],
        out_shape=[jax.ShapeDtypeStruct((NP, H), jnp.float32),
                   jax.ShapeDtypeStruct((NP, 1), jnp.float32)],
    )(*args)


def _tc_alpha_max(hs2d, hd2d, vv2d, a2):
    BLK = 16

    def body(hs_ref, hd_ref, vv_ref, a_ref, o_ref):
        z = hs_ref[...] * a_ref[0, 0] + hd_ref[...] * a_ref[0, 1]
        al = jnp.maximum(z, 0.0) + 0.2 * jnp.minimum(z, 0.0)
        al = jnp.where(vv_ref[...] > 0, al, NEG)
        blkmax = jnp.max(al)

        @pl.when(pl.program_id(0) == 0)
        def _():
            o_ref[...] = jnp.full((1, 1), NEG, jnp.float32)

        o_ref[...] = jnp.maximum(o_ref[...], blkmax)

    return pl.pallas_call(
        body,
        grid=(EP // 128 // BLK,),
        in_specs=[pl.BlockSpec((BLK, 128), lambda i: (i, 0)),
                  pl.BlockSpec((BLK, 128), lambda i: (i, 0)),
                  pl.BlockSpec((BLK, 128), lambda i: (i, 0)),
                  pl.BlockSpec((1, 2), lambda i: (0, 0))],
        out_specs=pl.BlockSpec((1, 1), lambda i: (0, 0)),
        out_shape=jax.ShapeDtypeStruct((1, 1), jnp.float32),
    )(hs2d, hd2d, vv2d, a2)


def _tc_exp_rows(hs1, hd1, vv1, a2, M):
    BLK = 1024

    def body(hs_ref, hd_ref, vv_ref, a_ref, m_ref, o_ref):
        hs = hs_ref[...]
        z = hs * a_ref[0, 0] + hd_ref[...] * a_ref[0, 1]
        al = jnp.maximum(z, 0.0) + 0.2 * jnp.minimum(z, 0.0)
        ex = jnp.where(vv_ref[...] > 0, jnp.exp(al - m_ref[0, 0]), 0.0)
        lane = lax.broadcasted_iota(jnp.int32, (1, 128), 1)
        o_ref[...] = ex * (lane == 0) + (ex * hs) * (lane == 1)

    return pl.pallas_call(
        body,
        grid=(EP // BLK,),
        in_specs=[pl.BlockSpec((BLK, 1), lambda i: (i, 0)),
                  pl.BlockSpec((BLK, 1), lambda i: (i, 0)),
                  pl.BlockSpec((BLK, 1), lambda i: (i, 0)),
                  pl.BlockSpec((1, 2), lambda i: (0, 0)),
                  pl.BlockSpec((1, 1), lambda i: (0, 0))],
        out_specs=pl.BlockSpec((BLK, 128), lambda i: (i, 0)),
        out_shape=jax.ShapeDtypeStruct((EP, 128), jnp.float32),
    )(hs1, hd1, vv1, a2, M)


def _tc_score_topk(dn, m, x, k):
    def body(dn_ref, m_ref, x_ref, mnew_ref, xm_ref, lp_ref):
        den = dn_ref[0, :, 0:1] + dn_ref[1, :, 0:1]
        num = dn_ref[0, :, 1:2] + dn_ref[1, :, 1:2]
        score = num / jnp.maximum(den, 1e-16)
        sm = jnp.where(m_ref[...] > 0, score, NEG)
        u = lax.bitcast_convert_type(sm, jnp.uint32)
        u = jnp.where((u >> 31) == 0, u | jnp.uint32(0x80000000), ~u)

        def tstep(i, t):
            t2 = t | (jnp.uint32(1) << (jnp.uint32(31) - jnp.uint32(i)))
            cnt = jnp.sum((u >= t2).astype(jnp.int32))
            return jnp.where(cnt >= k, t2, t)

        t = lax.fori_loop(0, 32, tstep, jnp.uint32(0))
        g = jnp.sum((u > t).astype(jnp.int32))
        r = k - g
        idx = lax.broadcasted_iota(jnp.int32, (NP, 1), 0)

        def jstep(i, lohi):
            lo, hi = lohi
            mid = (lo + hi) // 2
            cnt = jnp.sum(((u == t) & (idx < mid)).astype(jnp.int32))
            return (jnp.where(cnt >= r, lo, mid), jnp.where(cnt >= r, mid, hi))

        _, j = lax.fori_loop(0, 15, jstep, (jnp.int32(0), jnp.int32(NP)))
        mnew = ((u > t) | ((u == t) & (idx < j))).astype(jnp.float32)
        mnew_ref[...] = mnew
        xm = x_ref[...] * jnp.tanh(score) * mnew
        xm_ref[...] = xm
        gmp = jnp.max(jnp.where(mnew > 0, xm, NEG), axis=0, keepdims=True)
        gap = jnp.sum(xm, axis=0, keepdims=True) * (1.0 / k)
        lp_ref[:, 0:128] = gmp
        lp_ref[:, 128:256] = gap

    return pl.pallas_call(
        body,
        out_shape=[jax.ShapeDtypeStruct((NP, 1), jnp.float32),
                   jax.ShapeDtypeStruct((NP, H), jnp.float32),
                   jax.ShapeDtypeStruct((1, 256), jnp.float32)],
    )(dn, m, x)


def _tc_mlp(l1, l2, l3, W1, b1, W2, b2, W3, b3):
    def body(l1_ref, l2_ref, l3_ref, w1_ref, b1_ref, w2_ref, b2_ref,
             w3_ref, b3_ref, o_ref):
        l = l1_ref[...] + l2_ref[...] + l3_ref[...]
        h = jnp.maximum(jnp.dot(l, w1_ref[...], preferred_element_type=jnp.float32)
                        + b1_ref[...], 0.0)
        h = jnp.maximum(jnp.dot(h, w2_ref[...], preferred_element_type=jnp.float32)
                        + b2_ref[...], 0.0)
        o_ref[...] = jnp.dot(h, w3_ref[...], preferred_element_type=jnp.float32) + b3_ref[...]

    return pl.pallas_call(
        body,
        out_shape=jax.ShapeDtypeStruct((1, W3.shape[1]), jnp.float32),
    )(l1, l2, l3, W1, b1.reshape(1, -1), W2, b2.reshape(1, -1), W3, b3.reshape(1, -1))


def kernel(node_feat, node_level, edge_index, edge_feat, W1, b1, W2, b2, W3, b3, W4, b4, gat2_W, gat2_a, gat3_W, gat3_a, gat4_W, gat4_a, Wl1, bl1, Wl2, bl2, Wl3, bl3):
    f32 = jnp.float32
    src = edge_index[0]
    dst = edge_index[1]
    src1d = jnp.concatenate([src, jnp.zeros((EP - E,), jnp.int32)])
    dst1d = jnp.concatenate([dst, jnp.full((EP - E,), DUMP, jnp.int32)])
    src2d = src1d.reshape(EP // 128, 128)
    dst2d = dst1d.reshape(EP // 128, 128)
    efp = jnp.concatenate([edge_feat, jnp.zeros((EP - E, 4), f32)])
    x0 = jnp.concatenate([node_feat.reshape(N, -1), node_level], axis=1)
    x0 = jnp.concatenate([x0, jnp.zeros((NP - N, x0.shape[1]), f32)])
    zeros128 = jnp.zeros((NP, 128), f32)
    ones_row = jnp.ones((CHUNK, 128), f32)
    m1 = jnp.concatenate([jnp.ones((N, 1), f32), jnp.zeros((NP - N, 1), f32)])

    def conv(x, m, W, b, linprev, relu_prev, gatW, with_gd=True, deg=None,
             xd_corr=False):
        Fin = x.shape[1]
        Ws, Wd, We = W[:Fin], W[Fin:2 * Fin], W[2 * Fin:]
        xs, xd = _tc_prep(x, m, Ws, Wd, b)
        gs = _sc_gather128(xs, src2d)
        gd = _sc_gather128(xd, dst2d) if with_gd else None
        lin, dstp = _tc_edge(gs, gd, linprev, dst2d, We, relu_prev)
        acc = _sc_scatter128(lin, dstp, zeros128)
        if deg is None:
            deg = _sc_deg(dstp, ones_row, zeros128)
        x_next, h = _tc_fin(acc, deg, xd if xd_corr else None, gatW)
        return x_next, h, lin, dstp, deg

    def gat_topk(h, m, x, ga, k):
        hs, hd, vv = _sc_gat_gather(h.reshape(NP), m.reshape(NP), src1d, dst1d)
        a2 = ga.reshape(1, 2)
        M = _tc_alpha_max(hs.reshape(EP // 128, 128), hd.reshape(EP // 128, 128),
                          vv.reshape(EP // 128, 128), a2)
        rows = _tc_exp_rows(hs.reshape(EP, 1), hd.reshape(EP, 1),
                            vv.reshape(EP, 1), a2, M)
        dn = _sc_scatter128(rows, dst2d, zeros128)
        return _tc_score_topk(dn, m, x, k)

    x1, _, lin1, _, deg1 = conv(x0, None, W1[:44], b1, efp, False, gat2_W)
    x2, h2, lin2, _, _ = conv(x1, None, W2, b2, lin1, True, gat2_W, deg=deg1)
    m2, xm2, l1p = gat_topk(h2, m1, x2, gat2_a, 5000)
    x3, h3, lin3, _, _ = conv(xm2, m2, W3, b3, lin2, True, gat3_W)
    m3, xm3, l2p = gat_topk(h3, m2, x3, gat3_a, 2500)
    x4, h4, _, _, _ = conv(xm3, m3, W4, b4, lin3, True, gat4_W,
                           with_gd=False, xd_corr=True)
    _, _, l3p = gat_topk(h4, m3, x4, gat4_a, 1250)
    return _tc_mlp(l1p, l2p, l3p, Wl1, bl1, Wl2, bl2, Wl3, bl3)


# speedup vs baseline: 5.8533x; 1.0318x over previous
import functools
import math

import jax
import jax.numpy as jnp
from jax import lax
from jax.experimental import pallas as pl
from jax.experimental.pallas import tpu as pltpu
from jax.experimental.pallas import tpu_sc as plsc

N = 10000
E = 320000
H = 128
NP = 10240
DUMP = 10000
NW = 32
CHUNK = 128
CPW = 80
EPW = CPW * CHUNK
EP = NW * EPW
NEG = -1e30
NROW = NP // 16

_mesh = plsc.VectorSubcoreMesh(core_axis_name="c", subcore_axis_name="s")


def _sc_gather128(table, idx2d):
    @functools.partial(
        pl.kernel, mesh=_mesh,
        out_type=jax.ShapeDtypeStruct((EP, 128), jnp.float32),
        scratch_types=[pltpu.VMEM((CPW, CHUNK), jnp.int32),
                       pltpu.VMEM((6, CHUNK, 128), jnp.float32),
                       pltpu.SemaphoreType.DMA,
                       pltpu.SemaphoreType.DMA],
    )
    def k(table_hbm, idx_hbm, out_hbm, idx_v, rbuf, sem_g, sem_o):
        wid = lax.axis_index("s") * 2 + lax.axis_index("c")
        pltpu.sync_copy(idx_hbm.at[pl.ds(wid * CPW, CPW)], idx_v)
        base = wid * EPW
        gh, oh = {}, {}
        for j in range(CPW + 3):
            if j < CPW:
                if j >= 6:
                    oh[j - 6].wait()
                gh[j] = pltpu.async_copy(table_hbm.at[idx_v.at[j]],
                                         rbuf.at[j % 6], sem_g)
            if j >= 3:
                i = j - 3
                gh[i].wait()
                oh[i] = pltpu.async_copy(
                    rbuf.at[i % 6],
                    out_hbm.at[pl.ds(base + i * CHUNK, CHUNK)], sem_o)
        for i in range(CPW - 6, CPW):
            oh[i].wait()

    return k(table, idx2d)


def _sc_scatter128(rows, idx2d, zeros):
    @functools.partial(
        pl.kernel, mesh=_mesh,
        out_type=jax.ShapeDtypeStruct((2, NP, 128), jnp.float32),
        scratch_types=[pltpu.VMEM((CPW, CHUNK), jnp.int32),
                       pltpu.VMEM((2, CHUNK, 128), jnp.float32),
                       pltpu.VMEM_SHARED((NP, 128), jnp.float32),
                       pltpu.SemaphoreType.DMA,
                       pltpu.SemaphoreType.DMA],
    )
    def k(rows_hbm, idx_hbm, zeros_hbm, out_hbm, idx_v, rbuf, shared, sem_l, sem_s):
        cid = lax.axis_index("c")
        sid = lax.axis_index("s")
        wid = sid * 2 + cid
        pltpu.sync_copy(zeros_hbm.at[pl.ds(sid * NROW, NROW)],
                        shared.at[pl.ds(sid * NROW, NROW)])
        pltpu.sync_copy(idx_hbm.at[pl.ds(wid * CPW, CPW)], idx_v)
        plsc.subcore_barrier()
        base = wid * EPW
        lh, sh = {}, {}
        for t in range(2):
            lh[t] = pltpu.async_copy(rows_hbm.at[pl.ds(base + t * CHUNK, CHUNK)],
                                     rbuf.at[t], sem_l)
        for j in range(CPW):
            lh[j].wait()
            sh[j] = pltpu.async_copy(rbuf.at[j % 2], shared.at[idx_v.at[j]],
                                     sem_s, add=True)
            sh[j].wait()
            if j + 2 < CPW:
                lh[j + 2] = pltpu.async_copy(
                    rows_hbm.at[pl.ds(base + (j + 2) * CHUNK, CHUNK)],
                    rbuf.at[j % 2], sem_l)
        plsc.subcore_barrier()
        pltpu.sync_copy(shared.at[pl.ds(sid * NROW, NROW)],
                        out_hbm.at[cid].at[pl.ds(sid * NROW, NROW)])

    return k(rows, idx2d, zeros)


def _sc_deg(idx2d, ones_row, zeros128):
    @functools.partial(
        pl.kernel, mesh=_mesh,
        out_type=jax.ShapeDtypeStruct((2, NP, 128), jnp.float32),
        scratch_types=[pltpu.VMEM((CPW, CHUNK), jnp.int32),
                       pltpu.VMEM((CHUNK, 128), jnp.float32),
                       pltpu.VMEM_SHARED((NP, 128), jnp.float32),
                       pltpu.SemaphoreType.DMA],
    )
    def k(idx_hbm, ones_hbm, zeros_hbm, out_hbm, idx_v, ones_v, shared, sem_s):
        cid = lax.axis_index("c")
        sid = lax.axis_index("s")
        wid = sid * 2 + cid
        pltpu.sync_copy(zeros_hbm.at[pl.ds(sid * NROW, NROW)],
                        shared.at[pl.ds(sid * NROW, NROW)])
        pltpu.sync_copy(idx_hbm.at[pl.ds(wid * CPW, CPW)], idx_v)
        pltpu.sync_copy(ones_hbm, ones_v)
        plsc.subcore_barrier()
        sh = {}
        for j in range(CPW):
            sh[j] = pltpu.async_copy(ones_v, shared.at[idx_v.at[j]],
                                     sem_s, add=True)
            if j >= 4:
                sh[j - 4].wait()
        for j in range(CPW - 4, CPW):
            sh[j].wait()
        plsc.subcore_barrier()
        pltpu.sync_copy(shared.at[pl.ds(sid * NROW, NROW)],
                        out_hbm.at[cid].at[pl.ds(sid * NROW, NROW)])

    return k(idx2d, ones_row, zeros128)


def _sc_gat_gather(h, m, src1d, dst1d):
    @functools.partial(
        pl.kernel, mesh=_mesh,
        compiler_params=pltpu.CompilerParams(needs_layout_passes=False),
        out_type=[jax.ShapeDtypeStruct((EP,), jnp.float32),
                  jax.ShapeDtypeStruct((EP,), jnp.float32),
                  jax.ShapeDtypeStruct((EP,), jnp.float32)],
        scratch_types=[pltpu.VMEM((NP,), jnp.float32),
                       pltpu.VMEM((NP,), jnp.float32),
                       pltpu.VMEM((EPW,), jnp.int32),
                       pltpu.VMEM((EPW,), jnp.int32),
                       pltpu.VMEM((EPW,), jnp.float32),
                       pltpu.VMEM((EPW,), jnp.float32),
                       pltpu.VMEM((EPW,), jnp.float32)],
    )
    def k(h_hbm, m_hbm, src_hbm, dst_hbm, hs_hbm, hd_hbm, vv_hbm,
          h_v, m_v, src_v, dst_v, hs_v, hd_v, vv_v):
        wid = lax.axis_index("s") * 2 + lax.axis_index("c")
        base = wid * EPW
        pltpu.sync_copy(h_hbm, h_v)
        pltpu.sync_copy(m_hbm, m_v)
        pltpu.sync_copy(src_hbm.at[pl.ds(base, EPW)], src_v)
        pltpu.sync_copy(dst_hbm.at[pl.ds(base, EPW)], dst_v)

        def body(g, carry):
            o = g * 16
            is_ = src_v[pl.ds(o, 16)]
            id_ = dst_v[pl.ds(o, 16)]
            hs = plsc.load_gather(h_v, [is_])
            hd = plsc.load_gather(h_v, [id_])
            ms = plsc.load_gather(m_v, [is_])
            md = plsc.load_gather(m_v, [id_])
            hs_v[pl.ds(o, 16)] = hs
            hd_v[pl.ds(o, 16)] = hd
            vv_v[pl.ds(o, 16)] = ms * md
            return carry

        lax.fori_loop(0, EPW // 16, body, 0)
        pltpu.sync_copy(hs_v, hs_hbm.at[pl.ds(base, EPW)])
        pltpu.sync_copy(hd_v, hd_hbm.at[pl.ds(base, EPW)])
        pltpu.sync_copy(vv_v, vv_hbm.at[pl.ds(base, EPW)])

    return k(h, m, src1d, dst1d)


def _tc_prep(x, m, Ws, Wd, b):
    Fin = x.shape[1]
    BLK = 1024
    masked = m is not None

    def body(*refs):
        if masked:
            x_ref, m_ref, ws_ref, wd_ref, b_ref, xs_ref, xd_ref = refs
        else:
            x_ref, ws_ref, wd_ref, b_ref, xs_ref, xd_ref = refs
        xb = x_ref[...]
        xs = jnp.dot(xb, ws_ref[...], preferred_element_type=jnp.float32)
        xd = jnp.dot(xb, wd_ref[...], preferred_element_type=jnp.float32) + b_ref[...]
        if masked:
            keep = m_ref[...] > 0
            xs = jnp.where(keep, xs, NEG)
            xd = jnp.where(keep, xd, NEG)
        xs_ref[...] = xs
        xd_ref[...] = xd

    in_specs = [pl.BlockSpec((BLK, Fin), lambda i: (i, 0))]
    args = [x]
    if masked:
        in_specs.append(pl.BlockSpec((BLK, 1), lambda i: (i, 0)))
        args.append(m)
    in_specs += [pl.BlockSpec((Fin, H), lambda i: (0, 0)),
                 pl.BlockSpec((Fin, H), lambda i: (0, 0)),
                 pl.BlockSpec((1, H), lambda i: (0, 0))]
    args += [Ws, Wd, b.reshape(1, H)]
    return pl.pallas_call(
        body,
        grid=(NP // BLK,),
        in_specs=in_specs,
        out_specs=[pl.BlockSpec((BLK, H), lambda i: (i, 0)),
                   pl.BlockSpec((BLK, H), lambda i: (i, 0))],
        out_shape=[jax.ShapeDtypeStruct((NP, H), jnp.float32),
                   jax.ShapeDtypeStruct((NP, H), jnp.float32)],
    )(*args)


def _tc_edge(gs, gd, linprev, dst2d, We, relu_prev):
    BLK = 1024
    Dp = linprev.shape[1]
    with_gd = gd is not None

    def body(*refs):
        if with_gd:
            gs_ref, gd_ref, lp_ref, dst_ref, we_ref, lin_ref, dstp_ref = refs
        else:
            gs_ref, lp_ref, dst_ref, we_ref, lin_ref, dstp_ref = refs
        ep = lp_ref[...]
        if relu_prev:
            ep = jnp.maximum(ep, 0.0)
        mm = jnp.dot(ep, we_ref[...], preferred_element_type=jnp.float32)
        gsb = gs_ref[...]
        ok = gsb[:, 0:1] > -1e29
        if with_gd:
            gdb = gd_ref[...]
            ok = jnp.logical_and(ok, gdb[:, 0:1] > -1e29)
            lin_ref[...] = gsb + gdb + mm
        else:
            lin_ref[...] = gsb + mm
        okr = jnp.reshape(ok, (BLK // 128, 128))
        dstp_ref[...] = jnp.where(okr, dst_ref[...], DUMP)

    in_specs = [pl.BlockSpec((BLK, 128), lambda i: (i, 0))]
    args = [gs]
    if with_gd:
        in_specs.append(pl.BlockSpec((BLK, 128), lambda i: (i, 0)))
        args.append(gd)
    in_specs += [pl.BlockSpec((BLK, Dp), lambda i: (i, 0)),
                 pl.BlockSpec((BLK // 128, 128), lambda i: (i, 0)),
                 pl.BlockSpec((Dp, H), lambda i: (0, 0))]
    args += [linprev, dst2d, We]
    return pl.pallas_call(
        body,
        grid=(EP // BLK,),
        in_specs=in_specs,
        out_specs=[pl.BlockSpec((BLK, 128), lambda i: (i, 0)),
                   pl.BlockSpec((BLK // 128, 128), lambda i: (i, 0))],
        out_shape=[jax.ShapeDtypeStruct((EP, 128), jnp.float32),
                   jax.ShapeDtypeStruct((EP // 128, 128), jnp.int32)],
    )(*args)


def _tc_fin(acc, deg, xdLb, gatW):
    BLK = 1024
    with_xd = xdLb is not None

    def body(*refs):
        if with_xd:
            a_ref, d_ref, xd_ref, gw_ref, x_ref, h_ref = refs
        else:
            a_ref, d_ref, gw_ref, x_ref, h_ref = refs
        acc_b = a_ref[0] + a_ref[1]
        deg_b = jnp.maximum(d_ref[0][:, 0:1] + d_ref[1][:, 0:1], 1.0)
        xb = acc_b / deg_b
        if with_xd:
            xb = xb + xd_ref[...]
        xb = jnp.maximum(xb, 0.0)
        x_ref[...] = xb
        h_ref[...] = jnp.dot(xb, gw_ref[...], preferred_element_type=jnp.float32)

    in_specs = [pl.BlockSpec((2, BLK, H), lambda i: (0, i, 0)),
                pl.BlockSpec((2, BLK, H), lambda i: (0, i, 0))]
    args = [acc, deg]
    if with_xd:
        in_specs.append(pl.BlockSpec((BLK, H), lambda i: (i, 0)))
        args.append(xdLb)
    in_specs.append(pl.BlockSpec((H, 1), lambda i: (0, 0)))
    args.append(gatW)
    return pl.pallas_call(
        body,
        grid=(NP // BLK,),
        in_specs=in_specs,
        out_specs=[pl.BlockSpec((BLK, H), lambda i: (i, 0)),
                   pl.BlockSpec((BLK, 1), lambda i: (i, 0))],
        out_shape=[jax.ShapeDtypeStruct((NP, H), jnp.float32),
                   jax.ShapeDtypeStruct((NP, 1), jnp.float32)],
    )(*args)


def _tc_alpha_max(hs2d, hd2d, vv2d, a2):
    BLK = 16

    def body(hs_ref, hd_ref, vv_ref, a_ref, o_ref):
        z = hs_ref[...] * a_ref[0, 0] + hd_ref[...] * a_ref[0, 1]
        al = jnp.maximum(z, 0.0) + 0.2 * jnp.minimum(z, 0.0)
        al = jnp.where(vv_ref[...] > 0, al, NEG)
        blkmax = jnp.max(al)

        @pl.when(pl.program_id(0) == 0)
        def _():
            o_ref[...] = jnp.full((1, 1), NEG, jnp.float32)

        o_ref[...] = jnp.maximum(o_ref[...], blkmax)

    return pl.pallas_call(
        body,
        grid=(EP // 128 // BLK,),
        in_specs=[pl.BlockSpec((BLK, 128), lambda i: (i, 0)),
                  pl.BlockSpec((BLK, 128), lambda i: (i, 0)),
                  pl.BlockSpec((BLK, 128), lambda i: (i, 0)),
                  pl.BlockSpec((1, 2), lambda i: (0, 0))],
        out_specs=pl.BlockSpec((1, 1), lambda i: (0, 0)),
        out_shape=jax.ShapeDtypeStruct((1, 1), jnp.float32),
    )(hs2d, hd2d, vv2d, a2)


def _tc_exp_rows(hs1, hd1, vv1, a2, M):
    BLK = 1024

    def body(hs_ref, hd_ref, vv_ref, a_ref, m_ref, o_ref):
        hs = hs_ref[...]
        z = hs * a_ref[0, 0] + hd_ref[...] * a_ref[0, 1]
        al = jnp.maximum(z, 0.0) + 0.2 * jnp.minimum(z, 0.0)
        ex = jnp.where(vv_ref[...] > 0, jnp.exp(al - m_ref[0, 0]), 0.0)
        lane = lax.broadcasted_iota(jnp.int32, (1, 128), 1)
        o_ref[...] = ex * (lane == 0) + (ex * hs) * (lane == 1)

    return pl.pallas_call(
        body,
        grid=(EP // BLK,),
        in_specs=[pl.BlockSpec((BLK, 1), lambda i: (i, 0)),
                  pl.BlockSpec((BLK, 1), lambda i: (i, 0)),
                  pl.BlockSpec((BLK, 1), lambda i: (i, 0)),
                  pl.BlockSpec((1, 2), lambda i: (0, 0)),
                  pl.BlockSpec((1, 1), lambda i: (0, 0))],
        out_specs=pl.BlockSpec((BLK, 128), lambda i: (i, 0)),
        out_shape=jax.ShapeDtypeStruct((EP, 128), jnp.float32),
    )(hs1, hd1, vv1, a2, M)


def _tc_score_topk(dn, m, x, k):
    def body(dn_ref, m_ref, x_ref, mnew_ref, xm_ref, lp_ref):
        den = dn_ref[0, :, 0:1] + dn_ref[1, :, 0:1]
        num = dn_ref[0, :, 1:2] + dn_ref[1, :, 1:2]
        score = num / jnp.maximum(den, 1e-16)
        sm = jnp.where(m_ref[...] > 0, score, NEG)
        u = lax.bitcast_convert_type(sm, jnp.uint32)
        u = jnp.where((u >> 31) == 0, u | jnp.uint32(0x80000000), ~u)

        def tstep(i, t):
            t2 = t | (jnp.uint32(1) << (jnp.uint32(31) - jnp.uint32(i)))
            cnt = jnp.sum((u >= t2).astype(jnp.int32))
            return jnp.where(cnt >= k, t2, t)

        t = lax.fori_loop(0, 32, tstep, jnp.uint32(0))
        g = jnp.sum((u > t).astype(jnp.int32))
        r = k - g
        idx = lax.broadcasted_iota(jnp.int32, (NP, 1), 0)

        def jstep(i, lohi):
            lo, hi = lohi
            mid = (lo + hi) // 2
            cnt = jnp.sum(((u == t) & (idx < mid)).astype(jnp.int32))
            return (jnp.where(cnt >= r, lo, mid), jnp.where(cnt >= r, mid, hi))

        _, j = lax.fori_loop(0, 15, jstep, (jnp.int32(0), jnp.int32(NP)))
        mnew = ((u > t) | ((u == t) & (idx < j))).astype(jnp.float32)
        mnew_ref[...] = mnew
        xm = x_ref[...] * jnp.tanh(score) * mnew
        xm_ref[...] = xm
        gmp = jnp.max(jnp.where(mnew > 0, xm, NEG), axis=0, keepdims=True)
        gap = jnp.sum(xm, axis=0, keepdims=True) * (1.0 / k)
        lp_ref[:, 0:128] = gmp
        lp_ref[:, 128:256] = gap

    return pl.pallas_call(
        body,
        out_shape=[jax.ShapeDtypeStruct((NP, 1), jnp.float32),
                   jax.ShapeDtypeStruct((NP, H), jnp.float32),
                   jax.ShapeDtypeStruct((1, 256), jnp.float32)],
    )(dn, m, x)


def _tc_mlp(l1, l2, l3, W1, b1, W2, b2, W3, b3):
    def body(l1_ref, l2_ref, l3_ref, w1_ref, b1_ref, w2_ref, b2_ref,
             w3_ref, b3_ref, o_ref):
        l = l1_ref[...] + l2_ref[...] + l3_ref[...]
        h = jnp.maximum(jnp.dot(l, w1_ref[...], preferred_element_type=jnp.float32)
                        + b1_ref[...], 0.0)
        h = jnp.maximum(jnp.dot(h, w2_ref[...], preferred_element_type=jnp.float32)
                        + b2_ref[...], 0.0)
        o_ref[...] = jnp.dot(h, w3_ref[...], preferred_element_type=jnp.float32) + b3_ref[...]

    return pl.pallas_call(
        body,
        out_shape=jax.ShapeDtypeStruct((1, W3.shape[1]), jnp.float32),
    )(l1, l2, l3, W1, b1.reshape(1, -1), W2, b2.reshape(1, -1), W3, b3.reshape(1, -1))


def kernel(node_feat, node_level, edge_index, edge_feat, W1, b1, W2, b2, W3, b3, W4, b4, gat2_W, gat2_a, gat3_W, gat3_a, gat4_W, gat4_a, Wl1, bl1, Wl2, bl2, Wl3, bl3):
    f32 = jnp.float32
    src = edge_index[0]
    dst = edge_index[1]
    src1d = jnp.concatenate([src, jnp.zeros((EP - E,), jnp.int32)])
    dst1d = jnp.concatenate([dst, jnp.full((EP - E,), DUMP, jnp.int32)])
    src2d = src1d.reshape(EP // 128, 128)
    dst2d = dst1d.reshape(EP // 128, 128)
    efp = jnp.concatenate([edge_feat, jnp.zeros((EP - E, 4), f32)])
    x0 = jnp.concatenate([node_feat.reshape(N, -1), node_level], axis=1)
    x0 = jnp.concatenate([x0, jnp.zeros((NP - N, x0.shape[1]), f32)])
    zeros128 = jnp.zeros((NP, 128), f32)
    ones_row = jnp.ones((CHUNK, 128), f32)
    m1 = jnp.concatenate([jnp.ones((N, 1), f32), jnp.zeros((NP - N, 1), f32)])

    def conv(x, m, W, b, linprev, relu_prev, gatW, with_gd=True, deg=None,
             xd_corr=False):
        Fin = x.shape[1]
        Ws, Wd, We = W[:Fin], W[Fin:2 * Fin], W[2 * Fin:]
        xs, xd = _tc_prep(x, m, Ws, Wd, b)
        gs = _sc_gather128(xs, src2d)
        gd = _sc_gather128(xd, dst2d) if with_gd else None
        lin, dstp = _tc_edge(gs, gd, linprev, dst2d, We, relu_prev)
        zdep = zeros128
        if deg is None:
            deg = _sc_deg(dstp, ones_row, zeros128)
            zdep = zeros128 + deg[0, :1, :1] * 0.0
        acc = _sc_scatter128(lin, dstp, zdep)
        x_next, h = _tc_fin(acc, deg, xd if xd_corr else None, gatW)
        return x_next, h, lin, dstp, deg

    def gat_topk(h, m, x, ga, k):
        hs, hd, vv = _sc_gat_gather(h.reshape(NP), m.reshape(NP), src1d, dst1d)
        a2 = ga.reshape(1, 2)
        M = _tc_alpha_max(hs.reshape(EP // 128, 128), hd.reshape(EP // 128, 128),
                          vv.reshape(EP // 128, 128), a2)
        rows = _tc_exp_rows(hs.reshape(EP, 1), hd.reshape(EP, 1),
                            vv.reshape(EP, 1), a2, M)
        dn = _sc_scatter128(rows, dst2d, zeros128)
        return _tc_score_topk(dn, m, x, k)

    x1, _, lin1, _, deg1 = conv(x0, None, W1[:44], b1, efp, False, gat2_W)
    x2, h2, lin2, _, _ = conv(x1, None, W2, b2, lin1, True, gat2_W, deg=deg1)
    m2, xm2, l1p = gat_topk(h2, m1, x2, gat2_a, 5000)
    x3, h3, lin3, _, _ = conv(xm2, m2, W3, b3, lin2, True, gat3_W)
    m3, xm3, l2p = gat_topk(h3, m2, x3, gat3_a, 2500)
    x4, h4, _, _, _ = conv(xm3, m3, W4, b4, lin3, True, gat4_W,
                           with_gd=False, xd_corr=True)
    _, _, l3p = gat_topk(h4, m3, x4, gat4_a, 1250)
    return _tc_mlp(l1p, l2p, l3p, Wl1, bl1, Wl2, bl2, Wl3, bl3)


# speedup vs baseline: 5.8596x; 1.0011x over previous
import functools
import math

import jax
import jax.numpy as jnp
from jax import lax
from jax.experimental import pallas as pl
from jax.experimental.pallas import tpu as pltpu
from jax.experimental.pallas import tpu_sc as plsc

N = 10000
E = 320000
H = 128
NP = 10240
DUMP = 10000
NW = 32
CHUNK = 128
CPW = 80
EPW = CPW * CHUNK
EP = NW * EPW
NEG = -1e30
NROW = NP // 16

_mesh = plsc.VectorSubcoreMesh(core_axis_name="c", subcore_axis_name="s")


def _sc_gather128(table, idx2d):
    @functools.partial(
        pl.kernel, mesh=_mesh,
        out_type=jax.ShapeDtypeStruct((EP, 128), jnp.float32),
        scratch_types=[pltpu.VMEM((CPW, CHUNK), jnp.int32),
                       pltpu.VMEM((7, CHUNK, 128), jnp.float32),
                       pltpu.SemaphoreType.DMA,
                       pltpu.SemaphoreType.DMA],
    )
    def k(table_hbm, idx_hbm, out_hbm, idx_v, rbuf, sem_g, sem_o):
        wid = lax.axis_index("s") * 2 + lax.axis_index("c")
        pltpu.sync_copy(idx_hbm.at[pl.ds(wid * CPW, CPW)], idx_v)
        base = wid * EPW
        gh, oh = {}, {}
        for j in range(CPW + 5):
            if j < CPW:
                if j >= 7:
                    oh[j - 7].wait()
                gh[j] = pltpu.async_copy(table_hbm.at[idx_v.at[j]],
                                         rbuf.at[j % 7], sem_g)
            if j >= 5:
                i = j - 5
                gh[i].wait()
                oh[i] = pltpu.async_copy(
                    rbuf.at[i % 7],
                    out_hbm.at[pl.ds(base + i * CHUNK, CHUNK)], sem_o)
        for i in range(CPW - 7, CPW):
            oh[i].wait()

    return k(table, idx2d)


def _sc_scatter128(rows, idx2d, zeros):
    @functools.partial(
        pl.kernel, mesh=_mesh,
        out_type=jax.ShapeDtypeStruct((2, NP, 128), jnp.float32),
        scratch_types=[pltpu.VMEM((CPW, CHUNK), jnp.int32),
                       pltpu.VMEM((2, CHUNK, 128), jnp.float32),
                       pltpu.VMEM_SHARED((NP, 128), jnp.float32),
                       pltpu.SemaphoreType.DMA,
                       pltpu.SemaphoreType.DMA],
    )
    def k(rows_hbm, idx_hbm, zeros_hbm, out_hbm, idx_v, rbuf, shared, sem_l, sem_s):
        cid = lax.axis_index("c")
        sid = lax.axis_index("s")
        wid = sid * 2 + cid
        pltpu.sync_copy(zeros_hbm.at[pl.ds(sid * NROW, NROW)],
                        shared.at[pl.ds(sid * NROW, NROW)])
        pltpu.sync_copy(idx_hbm.at[pl.ds(wid * CPW, CPW)], idx_v)
        plsc.subcore_barrier()
        base = wid * EPW
        lh, sh = {}, {}
        for t in range(2):
            lh[t] = pltpu.async_copy(rows_hbm.at[pl.ds(base + t * CHUNK, CHUNK)],
                                     rbuf.at[t], sem_l)
        for j in range(CPW):
            lh[j].wait()
            sh[j] = pltpu.async_copy(rbuf.at[j % 2], shared.at[idx_v.at[j]],
                                     sem_s, add=True)
            sh[j].wait()
            if j + 2 < CPW:
                lh[j + 2] = pltpu.async_copy(
                    rows_hbm.at[pl.ds(base + (j + 2) * CHUNK, CHUNK)],
                    rbuf.at[j % 2], sem_l)
        plsc.subcore_barrier()
        pltpu.sync_copy(shared.at[pl.ds(sid * NROW, NROW)],
                        out_hbm.at[cid].at[pl.ds(sid * NROW, NROW)])

    return k(rows, idx2d, zeros)


def _sc_deg(idx2d, ones_row, zeros128):
    @functools.partial(
        pl.kernel, mesh=_mesh,
        out_type=jax.ShapeDtypeStruct((2, NP, 128), jnp.float32),
        scratch_types=[pltpu.VMEM((CPW, CHUNK), jnp.int32),
                       pltpu.VMEM((CHUNK, 128), jnp.float32),
                       pltpu.VMEM_SHARED((NP, 128), jnp.float32),
                       pltpu.SemaphoreType.DMA],
    )
    def k(idx_hbm, ones_hbm, zeros_hbm, out_hbm, idx_v, ones_v, shared, sem_s):
        cid = lax.axis_index("c")
        sid = lax.axis_index("s")
        wid = sid * 2 + cid
        pltpu.sync_copy(zeros_hbm.at[pl.ds(sid * NROW, NROW)],
                        shared.at[pl.ds(sid * NROW, NROW)])
        pltpu.sync_copy(idx_hbm.at[pl.ds(wid * CPW, CPW)], idx_v)
        pltpu.sync_copy(ones_hbm, ones_v)
        plsc.subcore_barrier()
        sh = {}
        for j in range(CPW):
            sh[j] = pltpu.async_copy(ones_v, shared.at[idx_v.at[j]],
                                     sem_s, add=True)
            if j >= 4:
                sh[j - 4].wait()
        for j in range(CPW - 4, CPW):
            sh[j].wait()
        plsc.subcore_barrier()
        pltpu.sync_copy(shared.at[pl.ds(sid * NROW, NROW)],
                        out_hbm.at[cid].at[pl.ds(sid * NROW, NROW)])

    return k(idx2d, ones_row, zeros128)


def _sc_gat_gather(h, m, src1d, dst1d):
    @functools.partial(
        pl.kernel, mesh=_mesh,
        compiler_params=pltpu.CompilerParams(needs_layout_passes=False),
        out_type=[jax.ShapeDtypeStruct((EP,), jnp.float32),
                  jax.ShapeDtypeStruct((EP,), jnp.float32),
                  jax.ShapeDtypeStruct((EP,), jnp.float32)],
        scratch_types=[pltpu.VMEM((NP,), jnp.float32),
                       pltpu.VMEM((NP,), jnp.float32),
                       pltpu.VMEM((EPW,), jnp.int32),
                       pltpu.VMEM((EPW,), jnp.int32),
                       pltpu.VMEM((EPW,), jnp.float32),
                       pltpu.VMEM((EPW,), jnp.float32),
                       pltpu.VMEM((EPW,), jnp.float32)],
    )
    def k(h_hbm, m_hbm, src_hbm, dst_hbm, hs_hbm, hd_hbm, vv_hbm,
          h_v, m_v, src_v, dst_v, hs_v, hd_v, vv_v):
        wid = lax.axis_index("s") * 2 + lax.axis_index("c")
        base = wid * EPW
        pltpu.sync_copy(h_hbm, h_v)
        pltpu.sync_copy(m_hbm, m_v)
        pltpu.sync_copy(src_hbm.at[pl.ds(base, EPW)], src_v)
        pltpu.sync_copy(dst_hbm.at[pl.ds(base, EPW)], dst_v)

        def body(g, carry):
            o = g * 16
            is_ = src_v[pl.ds(o, 16)]
            id_ = dst_v[pl.ds(o, 16)]
            hs = plsc.load_gather(h_v, [is_])
            hd = plsc.load_gather(h_v, [id_])
            ms = plsc.load_gather(m_v, [is_])
            md = plsc.load_gather(m_v, [id_])
            hs_v[pl.ds(o, 16)] = hs
            hd_v[pl.ds(o, 16)] = hd
            vv_v[pl.ds(o, 16)] = ms * md
            return carry

        lax.fori_loop(0, EPW // 16, body, 0)
        pltpu.sync_copy(hs_v, hs_hbm.at[pl.ds(base, EPW)])
        pltpu.sync_copy(hd_v, hd_hbm.at[pl.ds(base, EPW)])
        pltpu.sync_copy(vv_v, vv_hbm.at[pl.ds(base, EPW)])

    return k(h, m, src1d, dst1d)


def _tc_prep(x, m, Ws, Wd, b):
    Fin = x.shape[1]
    BLK = 1024
    masked = m is not None

    def body(*refs):
        if masked:
            x_ref, m_ref, ws_ref, wd_ref, b_ref, xs_ref, xd_ref = refs
        else:
            x_ref, ws_ref, wd_ref, b_ref, xs_ref, xd_ref = refs
        xb = x_ref[...]
        xs = jnp.dot(xb, ws_ref[...], preferred_element_type=jnp.float32)
        xd = jnp.dot(xb, wd_ref[...], preferred_element_type=jnp.float32) + b_ref[...]
        if masked:
            keep = m_ref[...] > 0
            xs = jnp.where(keep, xs, NEG)
            xd = jnp.where(keep, xd, NEG)
        xs_ref[...] = xs
        xd_ref[...] = xd

    in_specs = [pl.BlockSpec((BLK, Fin), lambda i: (i, 0))]
    args = [x]
    if masked:
        in_specs.append(pl.BlockSpec((BLK, 1), lambda i: (i, 0)))
        args.append(m)
    in_specs += [pl.BlockSpec((Fin, H), lambda i: (0, 0)),
                 pl.BlockSpec((Fin, H), lambda i: (0, 0)),
                 pl.BlockSpec((1, H), lambda i: (0, 0))]
    args += [Ws, Wd, b.reshape(1, H)]
    return pl.pallas_call(
        body,
        grid=(NP // BLK,),
        in_specs=in_specs,
        out_specs=[pl.BlockSpec((BLK, H), lambda i: (i, 0)),
                   pl.BlockSpec((BLK, H), lambda i: (i, 0))],
        out_shape=[jax.ShapeDtypeStruct((NP, H), jnp.float32),
                   jax.ShapeDtypeStruct((NP, H), jnp.float32)],
    )(*args)


def _tc_edge(gs, gd, linprev, dst2d, We, relu_prev):
    BLK = 1024
    Dp = linprev.shape[1]
    with_gd = gd is not None

    def body(*refs):
        if with_gd:
            gs_ref, gd_ref, lp_ref, dst_ref, we_ref, lin_ref, dstp_ref = refs
        else:
            gs_ref, lp_ref, dst_ref, we_ref, lin_ref, dstp_ref = refs
        ep = lp_ref[...]
        if relu_prev:
            ep = jnp.maximum(ep, 0.0)
        mm = jnp.dot(ep, we_ref[...], preferred_element_type=jnp.float32)
        gsb = gs_ref[...]
        ok = gsb[:, 0:1] > -1e29
        if with_gd:
            gdb = gd_ref[...]
            ok = jnp.logical_and(ok, gdb[:, 0:1] > -1e29)
            lin_ref[...] = gsb + gdb + mm
        else:
            lin_ref[...] = gsb + mm
        okr = jnp.reshape(ok, (BLK // 128, 128))
        dstp_ref[...] = jnp.where(okr, dst_ref[...], DUMP)

    in_specs = [pl.BlockSpec((BLK, 128), lambda i: (i, 0))]
    args = [gs]
    if with_gd:
        in_specs.append(pl.BlockSpec((BLK, 128), lambda i: (i, 0)))
        args.append(gd)
    in_specs += [pl.BlockSpec((BLK, Dp), lambda i: (i, 0)),
                 pl.BlockSpec((BLK // 128, 128), lambda i: (i, 0)),
                 pl.BlockSpec((Dp, H), lambda i: (0, 0))]
    args += [linprev, dst2d, We]
    return pl.pallas_call(
        body,
        grid=(EP // BLK,),
        in_specs=in_specs,
        out_specs=[pl.BlockSpec((BLK, 128), lambda i: (i, 0)),
                   pl.BlockSpec((BLK // 128, 128), lambda i: (i, 0))],
        out_shape=[jax.ShapeDtypeStruct((EP, 128), jnp.float32),
                   jax.ShapeDtypeStruct((EP // 128, 128), jnp.int32)],
    )(*args)


def _tc_fin(acc, deg, xdLb, gatW):
    BLK = 1024
    with_xd = xdLb is not None

    def body(*refs):
        if with_xd:
            a_ref, d_ref, xd_ref, gw_ref, x_ref, h_ref = refs
        else:
            a_ref, d_ref, gw_ref, x_ref, h_ref = refs
        acc_b = a_ref[0] + a_ref[1]
        deg_b = jnp.maximum(d_ref[0][:, 0:1] + d_ref[1][:, 0:1], 1.0)
        xb = acc_b / deg_b
        if with_xd:
            xb = xb + xd_ref[...]
        xb = jnp.maximum(xb, 0.0)
        x_ref[...] = xb
        h_ref[...] = jnp.dot(xb, gw_ref[...], preferred_element_type=jnp.float32)

    in_specs = [pl.BlockSpec((2, BLK, H), lambda i: (0, i, 0)),
                pl.BlockSpec((2, BLK, H), lambda i: (0, i, 0))]
    args = [acc, deg]
    if with_xd:
        in_specs.append(pl.BlockSpec((BLK, H), lambda i: (i, 0)))
        args.append(xdLb)
    in_specs.append(pl.BlockSpec((H, 1), lambda i: (0, 0)))
    args.append(gatW)
    return pl.pallas_call(
        body,
        grid=(NP // BLK,),
        in_specs=in_specs,
        out_specs=[pl.BlockSpec((BLK, H), lambda i: (i, 0)),
                   pl.BlockSpec((BLK, 1), lambda i: (i, 0))],
        out_shape=[jax.ShapeDtypeStruct((NP, H), jnp.float32),
                   jax.ShapeDtypeStruct((NP, 1), jnp.float32)],
    )(*args)


def _tc_alpha_max(hs2d, hd2d, vv2d, a2):
    BLK = 16

    def body(hs_ref, hd_ref, vv_ref, a_ref, o_ref):
        z = hs_ref[...] * a_ref[0, 0] + hd_ref[...] * a_ref[0, 1]
        al = jnp.maximum(z, 0.0) + 0.2 * jnp.minimum(z, 0.0)
        al = jnp.where(vv_ref[...] > 0, al, NEG)
        blkmax = jnp.max(al)

        @pl.when(pl.program_id(0) == 0)
        def _():
            o_ref[...] = jnp.full((1, 1), NEG, jnp.float32)

        o_ref[...] = jnp.maximum(o_ref[...], blkmax)

    return pl.pallas_call(
        body,
        grid=(EP // 128 // BLK,),
        in_specs=[pl.BlockSpec((BLK, 128), lambda i: (i, 0)),
                  pl.BlockSpec((BLK, 128), lambda i: (i, 0)),
                  pl.BlockSpec((BLK, 128), lambda i: (i, 0)),
                  pl.BlockSpec((1, 2), lambda i: (0, 0))],
        out_specs=pl.BlockSpec((1, 1), lambda i: (0, 0)),
        out_shape=jax.ShapeDtypeStruct((1, 1), jnp.float32),
    )(hs2d, hd2d, vv2d, a2)


def _tc_exp_rows(hs1, hd1, vv1, a2, M):
    BLK = 1024

    def body(hs_ref, hd_ref, vv_ref, a_ref, m_ref, o_ref):
        hs = hs_ref[...]
        z = hs * a_ref[0, 0] + hd_ref[...] * a_ref[0, 1]
        al = jnp.maximum(z, 0.0) + 0.2 * jnp.minimum(z, 0.0)
        ex = jnp.where(vv_ref[...] > 0, jnp.exp(al - m_ref[0, 0]), 0.0)
        lane = lax.broadcasted_iota(jnp.int32, (1, 128), 1)
        o_ref[...] = ex * (lane == 0) + (ex * hs) * (lane == 1)

    return pl.pallas_call(
        body,
        grid=(EP // BLK,),
        in_specs=[pl.BlockSpec((BLK, 1), lambda i: (i, 0)),
                  pl.BlockSpec((BLK, 1), lambda i: (i, 0)),
                  pl.BlockSpec((BLK, 1), lambda i: (i, 0)),
                  pl.BlockSpec((1, 2), lambda i: (0, 0)),
                  pl.BlockSpec((1, 1), lambda i: (0, 0))],
        out_specs=pl.BlockSpec((BLK, 128), lambda i: (i, 0)),
        out_shape=jax.ShapeDtypeStruct((EP, 128), jnp.float32),
    )(hs1, hd1, vv1, a2, M)


def _tc_score_topk(dn, m, x, k):
    def body(dn_ref, m_ref, x_ref, mnew_ref, xm_ref, lp_ref):
        den = dn_ref[0, :, 0:1] + dn_ref[1, :, 0:1]
        num = dn_ref[0, :, 1:2] + dn_ref[1, :, 1:2]
        score = num / jnp.maximum(den, 1e-16)
        sm = jnp.where(m_ref[...] > 0, score, NEG)
        u = lax.bitcast_convert_type(sm, jnp.uint32)
        u = jnp.where((u >> 31) == 0, u | jnp.uint32(0x80000000), ~u)

        def tstep(i, t):
            t2 = t | (jnp.uint32(1) << (jnp.uint32(31) - jnp.uint32(i)))
            cnt = jnp.sum((u >= t2).astype(jnp.int32))
            return jnp.where(cnt >= k, t2, t)

        t = lax.fori_loop(0, 32, tstep, jnp.uint32(0))
        g = jnp.sum((u > t).astype(jnp.int32))
        r = k - g
        idx = lax.broadcasted_iota(jnp.int32, (NP, 1), 0)

        def jstep(i, lohi):
            lo, hi = lohi
            mid = (lo + hi) // 2
            cnt = jnp.sum(((u == t) & (idx < mid)).astype(jnp.int32))
            return (jnp.where(cnt >= r, lo, mid), jnp.where(cnt >= r, mid, hi))

        _, j = lax.fori_loop(0, 15, jstep, (jnp.int32(0), jnp.int32(NP)))
        mnew = ((u > t) | ((u == t) & (idx < j))).astype(jnp.float32)
        mnew_ref[...] = mnew
        xm = x_ref[...] * jnp.tanh(score) * mnew
        xm_ref[...] = xm
        gmp = jnp.max(jnp.where(mnew > 0, xm, NEG), axis=0, keepdims=True)
        gap = jnp.sum(xm, axis=0, keepdims=True) * (1.0 / k)
        lp_ref[:, 0:128] = gmp
        lp_ref[:, 128:256] = gap

    return pl.pallas_call(
        body,
        out_shape=[jax.ShapeDtypeStruct((NP, 1), jnp.float32),
                   jax.ShapeDtypeStruct((NP, H), jnp.float32),
                   jax.ShapeDtypeStruct((1, 256), jnp.float32)],
    )(dn, m, x)


def _tc_mlp(l1, l2, l3, W1, b1, W2, b2, W3, b3):
    def body(l1_ref, l2_ref, l3_ref, w1_ref, b1_ref, w2_ref, b2_ref,
             w3_ref, b3_ref, o_ref):
        l = l1_ref[...] + l2_ref[...] + l3_ref[...]
        h = jnp.maximum(jnp.dot(l, w1_ref[...], preferred_element_type=jnp.float32)
                        + b1_ref[...], 0.0)
        h = jnp.maximum(jnp.dot(h, w2_ref[...], preferred_element_type=jnp.float32)
                        + b2_ref[...], 0.0)
        o_ref[...] = jnp.dot(h, w3_ref[...], preferred_element_type=jnp.float32) + b3_ref[...]

    return pl.pallas_call(
        body,
        out_shape=jax.ShapeDtypeStruct((1, W3.shape[1]), jnp.float32),
    )(l1, l2, l3, W1, b1.reshape(1, -1), W2, b2.reshape(1, -1), W3, b3.reshape(1, -1))


def kernel(node_feat, node_level, edge_index, edge_feat, W1, b1, W2, b2, W3, b3, W4, b4, gat2_W, gat2_a, gat3_W, gat3_a, gat4_W, gat4_a, Wl1, bl1, Wl2, bl2, Wl3, bl3):
    f32 = jnp.float32
    src = edge_index[0]
    dst = edge_index[1]
    src1d = jnp.concatenate([src, jnp.zeros((EP - E,), jnp.int32)])
    dst1d = jnp.concatenate([dst, jnp.full((EP - E,), DUMP, jnp.int32)])
    src2d = src1d.reshape(EP // 128, 128)
    dst2d = dst1d.reshape(EP // 128, 128)
    efp = jnp.concatenate([edge_feat, jnp.zeros((EP - E, 4), f32)])
    x0 = jnp.concatenate([node_feat.reshape(N, -1), node_level], axis=1)
    x0 = jnp.concatenate([x0, jnp.zeros((NP - N, x0.shape[1]), f32)])
    zeros128 = jnp.zeros((NP, 128), f32)
    ones_row = jnp.ones((CHUNK, 128), f32)
    m1 = jnp.concatenate([jnp.ones((N, 1), f32), jnp.zeros((NP - N, 1), f32)])

    def conv(x, m, W, b, linprev, relu_prev, gatW, with_gd=True, deg=None,
             xd_corr=False):
        Fin = x.shape[1]
        Ws, Wd, We = W[:Fin], W[Fin:2 * Fin], W[2 * Fin:]
        xs, xd = _tc_prep(x, m, Ws, Wd, b)
        gs = _sc_gather128(xs, src2d)
        gd = _sc_gather128(xd, dst2d) if with_gd else None
        lin, dstp = _tc_edge(gs, gd, linprev, dst2d, We, relu_prev)
        zdep = zeros128
        if deg is None:
            deg = _sc_deg(dstp, ones_row, zeros128)
            zdep = zeros128 + deg[0, :1, :1] * 0.0
        acc = _sc_scatter128(lin, dstp, zdep)
        x_next, h = _tc_fin(acc, deg, xd if xd_corr else None, gatW)
        return x_next, h, lin, dstp, deg

    def gat_topk(h, m, x, ga, k):
        hs, hd, vv = _sc_gat_gather(h.reshape(NP), m.reshape(NP), src1d, dst1d)
        a2 = ga.reshape(1, 2)
        M = _tc_alpha_max(hs.reshape(EP // 128, 128), hd.reshape(EP // 128, 128),
                          vv.reshape(EP // 128, 128), a2)
        rows = _tc_exp_rows(hs.reshape(EP, 1), hd.reshape(EP, 1),
                            vv.reshape(EP, 1), a2, M)
        dn = _sc_scatter128(rows, dst2d, zeros128)
        return _tc_score_topk(dn, m, x, k)

    x1, _, lin1, _, deg1 = conv(x0, None, W1[:44], b1, efp, False, gat2_W)
    x2, h2, lin2, _, _ = conv(x1, None, W2, b2, lin1, True, gat2_W, deg=deg1)
    m2, xm2, l1p = gat_topk(h2, m1, x2, gat2_a, 5000)
    x3, h3, lin3, _, _ = conv(xm2, m2, W3, b3, lin2, True, gat3_W)
    m3, xm3, l2p = gat_topk(h3, m2, x3, gat3_a, 2500)
    x4, h4, _, _, _ = conv(xm3, m3, W4, b4, lin3, True, gat4_W,
                           with_gd=False, xd_corr=True)
    _, _, l3p = gat_topk(h4, m3, x4, gat4_a, 1250)
    return _tc_mlp(l1p, l2p, l3p, Wl1, bl1, Wl2, bl2, Wl3, bl3)


# speedup vs baseline: 7.6488x; 1.3053x over previous
import functools
import math

import jax
import jax.numpy as jnp
from jax import lax
from jax.experimental import pallas as pl
from jax.experimental.pallas import tpu as pltpu
from jax.experimental.pallas import tpu_sc as plsc

N = 10000
E = 320000
H = 128
NP = 10240
DUMP = 10000
NW = 32
CHUNK = 128
CPW = 80
EPW = CPW * CHUNK
EP = NW * EPW
NEG = -1e30
NROW = NP // 16

_mesh = plsc.VectorSubcoreMesh(core_axis_name="c", subcore_axis_name="s")


def _sc_gather128(table, idx2d):
    @functools.partial(
        pl.kernel, mesh=_mesh,
        out_type=jax.ShapeDtypeStruct((EP, 128), jnp.float32),
        scratch_types=[pltpu.VMEM((CPW, CHUNK), jnp.int32),
                       pltpu.VMEM((7, CHUNK, 128), jnp.float32),
                       pltpu.SemaphoreType.DMA,
                       pltpu.SemaphoreType.DMA],
    )
    def k(table_hbm, idx_hbm, out_hbm, idx_v, rbuf, sem_g, sem_o):
        wid = lax.axis_index("s") * 2 + lax.axis_index("c")
        pltpu.sync_copy(idx_hbm.at[pl.ds(wid * CPW, CPW)], idx_v)
        base = wid * EPW
        gh, oh = {}, {}
        for j in range(CPW + 5):
            if j < CPW:
                if j >= 7:
                    oh[j - 7].wait()
                gh[j] = pltpu.async_copy(table_hbm.at[idx_v.at[j]],
                                         rbuf.at[j % 7], sem_g)
            if j >= 5:
                i = j - 5
                gh[i].wait()
                oh[i] = pltpu.async_copy(
                    rbuf.at[i % 7],
                    out_hbm.at[pl.ds(base + i * CHUNK, CHUNK)], sem_o)
        for i in range(CPW - 7, CPW):
            oh[i].wait()

    return k(table, idx2d)


def _sc_scatter128(rows, idx2d, zeros):
    @functools.partial(
        pl.kernel, mesh=_mesh,
        out_type=jax.ShapeDtypeStruct((2, NP, 128), jnp.float32),
        scratch_types=[pltpu.VMEM((CPW, CHUNK), jnp.int32),
                       pltpu.VMEM((2, CHUNK, 128), jnp.float32),
                       pltpu.VMEM_SHARED((NP, 128), jnp.float32),
                       pltpu.SemaphoreType.DMA,
                       pltpu.SemaphoreType.DMA],
    )
    def k(rows_hbm, idx_hbm, zeros_hbm, out_hbm, idx_v, rbuf, shared, sem_l, sem_s):
        cid = lax.axis_index("c")
        sid = lax.axis_index("s")
        wid = sid * 2 + cid
        pltpu.sync_copy(zeros_hbm.at[pl.ds(sid * NROW, NROW)],
                        shared.at[pl.ds(sid * NROW, NROW)])
        pltpu.sync_copy(idx_hbm.at[pl.ds(wid * CPW, CPW)], idx_v)
        plsc.subcore_barrier()
        base = wid * EPW
        lh, sh = {}, {}
        for t in range(2):
            lh[t] = pltpu.async_copy(rows_hbm.at[pl.ds(base + t * CHUNK, CHUNK)],
                                     rbuf.at[t], sem_l)
        for j in range(CPW):
            lh[j].wait()
            sh[j] = pltpu.async_copy(rbuf.at[j % 2], shared.at[idx_v.at[j]],
                                     sem_s, add=True)
            sh[j].wait()
            if j + 2 < CPW:
                lh[j + 2] = pltpu.async_copy(
                    rows_hbm.at[pl.ds(base + (j + 2) * CHUNK, CHUNK)],
                    rbuf.at[j % 2], sem_l)
        plsc.subcore_barrier()
        pltpu.sync_copy(shared.at[pl.ds(sid * NROW, NROW)],
                        out_hbm.at[cid].at[pl.ds(sid * NROW, NROW)])

    return k(rows, idx2d, zeros)


def _sc_deg(idx1d, zerosN):
    @functools.partial(
        pl.kernel, mesh=_mesh,
        compiler_params=pltpu.CompilerParams(needs_layout_passes=False),
        out_type=jax.ShapeDtypeStruct((NW, NP), jnp.float32),
        scratch_types=[pltpu.VMEM((EPW,), jnp.int32),
                       pltpu.VMEM((NP,), jnp.float32)],
    )
    def k(idx_hbm, zeros_hbm, out_hbm, idx_v, acc_v):
        wid = lax.axis_index("s") * 2 + lax.axis_index("c")
        base = wid * EPW
        pltpu.sync_copy(idx_hbm.at[pl.ds(base, EPW)], idx_v)
        pltpu.sync_copy(zeros_hbm, acc_v)
        ones16 = jnp.zeros((16,), jnp.float32) + 1.0

        def body(g, carry):
            o = g * 16
            d16 = idx_v[pl.ds(o, 16)]
            plsc.addupdate_scatter(acc_v, [d16], ones16)
            return carry

        lax.fori_loop(0, EPW // 16, body, 0)
        pltpu.sync_copy(acc_v, out_hbm.at[wid])

    return k(idx1d, zerosN)


def _sc_gat_accum(hs, hd, vv, dst1d, a0bc, a1bc, M128, zerosN):
    @functools.partial(
        pl.kernel, mesh=_mesh,
        compiler_params=pltpu.CompilerParams(needs_layout_passes=False),
        out_type=[jax.ShapeDtypeStruct((NW, NP), jnp.float32),
                  jax.ShapeDtypeStruct((NW, NP), jnp.float32)],
        scratch_types=[pltpu.VMEM((EPW,), jnp.float32),
                       pltpu.VMEM((EPW,), jnp.float32),
                       pltpu.VMEM((EPW,), jnp.float32),
                       pltpu.VMEM((EPW,), jnp.int32),
                       pltpu.VMEM((16,), jnp.float32),
                       pltpu.VMEM((16,), jnp.float32),
                       pltpu.VMEM((128,), jnp.float32),
                       pltpu.VMEM((NP,), jnp.float32),
                       pltpu.VMEM((NP,), jnp.float32)],
    )
    def k(hs_hbm, hd_hbm, vv_hbm, dst_hbm, a0_hbm, a1_hbm, m_hbm, zeros_hbm,
          den_hbm, num_hbm, hs_v, hd_v, vv_v, dst_v, a0_v, a1_v, m_v,
          den_v, num_v):
        wid = lax.axis_index("s") * 2 + lax.axis_index("c")
        base = wid * EPW
        pltpu.sync_copy(hs_hbm.at[pl.ds(base, EPW)], hs_v)
        pltpu.sync_copy(hd_hbm.at[pl.ds(base, EPW)], hd_v)
        pltpu.sync_copy(vv_hbm.at[pl.ds(base, EPW)], vv_v)
        pltpu.sync_copy(dst_hbm.at[pl.ds(base, EPW)], dst_v)
        pltpu.sync_copy(a0_hbm, a0_v)
        pltpu.sync_copy(a1_hbm, a1_v)
        pltpu.sync_copy(m_hbm, m_v)
        pltpu.sync_copy(zeros_hbm, den_v)
        pltpu.sync_copy(zeros_hbm, num_v)
        a0 = a0_v[pl.ds(0, 16)]
        a1 = a1_v[pl.ds(0, 16)]
        M16 = m_v[pl.ds(0, 16)]

        def body(g, carry):
            o = g * 16
            hs16 = hs_v[pl.ds(o, 16)]
            hd16 = hd_v[pl.ds(o, 16)]
            vv16 = vv_v[pl.ds(o, 16)]
            d16 = dst_v[pl.ds(o, 16)]
            z = hs16 * a0 + hd16 * a1
            al = jnp.maximum(z, 0.0) + 0.2 * jnp.minimum(z, 0.0)
            ex = jnp.where(vv16 > 0, jnp.exp(al - M16), 0.0)
            plsc.addupdate_scatter(den_v, [d16], ex)
            plsc.addupdate_scatter(num_v, [d16], ex * hs16)
            return carry

        lax.fori_loop(0, EPW // 16, body, 0)
        pltpu.sync_copy(den_v, den_hbm.at[wid])
        pltpu.sync_copy(num_v, num_hbm.at[wid])

    return k(hs, hd, vv, dst1d, a0bc, a1bc, M128, zerosN)


def _sc_gat_gather(h, m, src1d, dst1d):
    @functools.partial(
        pl.kernel, mesh=_mesh,
        compiler_params=pltpu.CompilerParams(needs_layout_passes=False),
        out_type=[jax.ShapeDtypeStruct((EP,), jnp.float32),
                  jax.ShapeDtypeStruct((EP,), jnp.float32),
                  jax.ShapeDtypeStruct((EP,), jnp.float32)],
        scratch_types=[pltpu.VMEM((NP,), jnp.float32),
                       pltpu.VMEM((NP,), jnp.float32),
                       pltpu.VMEM((EPW,), jnp.int32),
                       pltpu.VMEM((EPW,), jnp.int32),
                       pltpu.VMEM((EPW,), jnp.float32),
                       pltpu.VMEM((EPW,), jnp.float32),
                       pltpu.VMEM((EPW,), jnp.float32)],
    )
    def k(h_hbm, m_hbm, src_hbm, dst_hbm, hs_hbm, hd_hbm, vv_hbm,
          h_v, m_v, src_v, dst_v, hs_v, hd_v, vv_v):
        wid = lax.axis_index("s") * 2 + lax.axis_index("c")
        base = wid * EPW
        pltpu.sync_copy(h_hbm, h_v)
        pltpu.sync_copy(m_hbm, m_v)
        pltpu.sync_copy(src_hbm.at[pl.ds(base, EPW)], src_v)
        pltpu.sync_copy(dst_hbm.at[pl.ds(base, EPW)], dst_v)

        def body(g, carry):
            o = g * 16
            is_ = src_v[pl.ds(o, 16)]
            id_ = dst_v[pl.ds(o, 16)]
            hs = plsc.load_gather(h_v, [is_])
            hd = plsc.load_gather(h_v, [id_])
            ms = plsc.load_gather(m_v, [is_])
            md = plsc.load_gather(m_v, [id_])
            hs_v[pl.ds(o, 16)] = hs
            hd_v[pl.ds(o, 16)] = hd
            vv_v[pl.ds(o, 16)] = ms * md
            return carry

        lax.fori_loop(0, EPW // 16, body, 0)
        pltpu.sync_copy(hs_v, hs_hbm.at[pl.ds(base, EPW)])
        pltpu.sync_copy(hd_v, hd_hbm.at[pl.ds(base, EPW)])
        pltpu.sync_copy(vv_v, vv_hbm.at[pl.ds(base, EPW)])

    return k(h, m, src1d, dst1d)


def _tc_prep(x, m, Ws, Wd, b):
    Fin = x.shape[1]
    BLK = 1024
    masked = m is not None

    def body(*refs):
        if masked:
            x_ref, m_ref, ws_ref, wd_ref, b_ref, xs_ref, xd_ref = refs
        else:
            x_ref, ws_ref, wd_ref, b_ref, xs_ref, xd_ref = refs
        xb = x_ref[...]
        xs = jnp.dot(xb, ws_ref[...], preferred_element_type=jnp.float32)
        xd = jnp.dot(xb, wd_ref[...], preferred_element_type=jnp.float32) + b_ref[...]
        if masked:
            keep = m_ref[...] > 0
            xs = jnp.where(keep, xs, NEG)
            xd = jnp.where(keep, xd, NEG)
        xs_ref[...] = xs
        xd_ref[...] = xd

    in_specs = [pl.BlockSpec((BLK, Fin), lambda i: (i, 0))]
    args = [x]
    if masked:
        in_specs.append(pl.BlockSpec((BLK, 1), lambda i: (i, 0)))
        args.append(m)
    in_specs += [pl.BlockSpec((Fin, H), lambda i: (0, 0)),
                 pl.BlockSpec((Fin, H), lambda i: (0, 0)),
                 pl.BlockSpec((1, H), lambda i: (0, 0))]
    args += [Ws, Wd, b.reshape(1, H)]
    return pl.pallas_call(
        body,
        grid=(NP // BLK,),
        in_specs=in_specs,
        out_specs=[pl.BlockSpec((BLK, H), lambda i: (i, 0)),
                   pl.BlockSpec((BLK, H), lambda i: (i, 0))],
        out_shape=[jax.ShapeDtypeStruct((NP, H), jnp.float32),
                   jax.ShapeDtypeStruct((NP, H), jnp.float32)],
    )(*args)


def _tc_edge(gs, gd, linprev, dst2d, We, relu_prev):
    BLK = 1024
    Dp = linprev.shape[1]
    with_gd = gd is not None

    def body(*refs):
        if with_gd:
            gs_ref, gd_ref, lp_ref, dst_ref, we_ref, lin_ref, dstp_ref = refs
        else:
            gs_ref, lp_ref, dst_ref, we_ref, lin_ref, dstp_ref = refs
        ep = lp_ref[...]
        if relu_prev:
            ep = jnp.maximum(ep, 0.0)
        mm = jnp.dot(ep, we_ref[...], preferred_element_type=jnp.float32)
        gsb = gs_ref[...]
        ok = gsb[:, 0:1] > -1e29
        if with_gd:
            gdb = gd_ref[...]
            ok = jnp.logical_and(ok, gdb[:, 0:1] > -1e29)
            lin_ref[...] = gsb + gdb + mm
        else:
            lin_ref[...] = gsb + mm
        okr = jnp.reshape(ok, (BLK // 128, 128))
        dstp_ref[...] = jnp.where(okr, dst_ref[...], DUMP)

    in_specs = [pl.BlockSpec((BLK, 128), lambda i: (i, 0))]
    args = [gs]
    if with_gd:
        in_specs.append(pl.BlockSpec((BLK, 128), lambda i: (i, 0)))
        args.append(gd)
    in_specs += [pl.BlockSpec((BLK, Dp), lambda i: (i, 0)),
                 pl.BlockSpec((BLK // 128, 128), lambda i: (i, 0)),
                 pl.BlockSpec((Dp, H), lambda i: (0, 0))]
    args += [linprev, dst2d, We]
    return pl.pallas_call(
        body,
        grid=(EP // BLK,),
        in_specs=in_specs,
        out_specs=[pl.BlockSpec((BLK, 128), lambda i: (i, 0)),
                   pl.BlockSpec((BLK // 128, 128), lambda i: (i, 0))],
        out_shape=[jax.ShapeDtypeStruct((EP, 128), jnp.float32),
                   jax.ShapeDtypeStruct((EP // 128, 128), jnp.int32)],
    )(*args)


def _tc_fin(acc, deg, xdLb, gatW):
    BLK = 1024
    with_xd = xdLb is not None

    def body(*refs):
        if with_xd:
            a_ref, d_ref, xd_ref, gw_ref, x_ref, h_ref = refs
        else:
            a_ref, d_ref, gw_ref, x_ref, h_ref = refs
        acc_b = a_ref[0] + a_ref[1]
        deg_b = jnp.maximum(jnp.sum(d_ref[...], axis=1, keepdims=True), 1.0)
        xb = acc_b / deg_b
        if with_xd:
            xb = xb + xd_ref[...]
        xb = jnp.maximum(xb, 0.0)
        x_ref[...] = xb
        h_ref[...] = jnp.dot(xb, gw_ref[...], preferred_element_type=jnp.float32)

    in_specs = [pl.BlockSpec((2, BLK, H), lambda i: (0, i, 0)),
                pl.BlockSpec((BLK, NW), lambda i: (i, 0))]
    args = [acc, deg]
    if with_xd:
        in_specs.append(pl.BlockSpec((BLK, H), lambda i: (i, 0)))
        args.append(xdLb)
    in_specs.append(pl.BlockSpec((H, 1), lambda i: (0, 0)))
    args.append(gatW)
    return pl.pallas_call(
        body,
        grid=(NP // BLK,),
        in_specs=in_specs,
        out_specs=[pl.BlockSpec((BLK, H), lambda i: (i, 0)),
                   pl.BlockSpec((BLK, 1), lambda i: (i, 0))],
        out_shape=[jax.ShapeDtypeStruct((NP, H), jnp.float32),
                   jax.ShapeDtypeStruct((NP, 1), jnp.float32)],
    )(*args)


def _tc_alpha_max(hs2d, hd2d, vv2d, a2):
    BLK = 16

    def body(hs_ref, hd_ref, vv_ref, a_ref, o_ref):
        z = hs_ref[...] * a_ref[0, 0] + hd_ref[...] * a_ref[0, 1]
        al = jnp.maximum(z, 0.0) + 0.2 * jnp.minimum(z, 0.0)
        al = jnp.where(vv_ref[...] > 0, al, NEG)
        blkmax = jnp.max(al)

        @pl.when(pl.program_id(0) == 0)
        def _():
            o_ref[...] = jnp.full((1, 128), NEG, jnp.float32)

        o_ref[...] = jnp.maximum(o_ref[...], blkmax)

    return pl.pallas_call(
        body,
        grid=(EP // 128 // BLK,),
        in_specs=[pl.BlockSpec((BLK, 128), lambda i: (i, 0)),
                  pl.BlockSpec((BLK, 128), lambda i: (i, 0)),
                  pl.BlockSpec((BLK, 128), lambda i: (i, 0)),
                  pl.BlockSpec((1, 2), lambda i: (0, 0))],
        out_specs=pl.BlockSpec((1, 128), lambda i: (0, 0)),
        out_shape=jax.ShapeDtypeStruct((1, 128), jnp.float32),
    )(hs2d, hd2d, vv2d, a2)


def _tc_score_topk(dn, nm, m, x, k):
    def body(dn_ref, nm_ref, m_ref, x_ref, mnew_ref, xm_ref, lp_ref):
        den = jnp.sum(dn_ref[...], axis=1, keepdims=True)
        num = jnp.sum(nm_ref[...], axis=1, keepdims=True)
        score = num / jnp.maximum(den, 1e-16)
        sm = jnp.where(m_ref[...] > 0, score, NEG)
        u = lax.bitcast_convert_type(sm, jnp.uint32)
        u = jnp.where((u >> 31) == 0, u | jnp.uint32(0x80000000), ~u)

        def tstep(i, t):
            t2 = t | (jnp.uint32(1) << (jnp.uint32(31) - jnp.uint32(i)))
            cnt = jnp.sum((u >= t2).astype(jnp.int32))
            return jnp.where(cnt >= k, t2, t)

        t = lax.fori_loop(0, 32, tstep, jnp.uint32(0))
        g = jnp.sum((u > t).astype(jnp.int32))
        r = k - g
        idx = lax.broadcasted_iota(jnp.int32, (NP, 1), 0)

        def jstep(i, lohi):
            lo, hi = lohi
            mid = (lo + hi) // 2
            cnt = jnp.sum(((u == t) & (idx < mid)).astype(jnp.int32))
            return (jnp.where(cnt >= r, lo, mid), jnp.where(cnt >= r, mid, hi))

        _, j = lax.fori_loop(0, 15, jstep, (jnp.int32(0), jnp.int32(NP)))
        mnew = ((u > t) | ((u == t) & (idx < j))).astype(jnp.float32)
        mnew_ref[...] = mnew
        xm = x_ref[...] * jnp.tanh(score) * mnew
        xm_ref[...] = xm
        gmp = jnp.max(jnp.where(mnew > 0, xm, NEG), axis=0, keepdims=True)
        gap = jnp.sum(xm, axis=0, keepdims=True) * (1.0 / k)
        lp_ref[:, 0:128] = gmp
        lp_ref[:, 128:256] = gap

    return pl.pallas_call(
        body,
        out_shape=[jax.ShapeDtypeStruct((NP, 1), jnp.float32),
                   jax.ShapeDtypeStruct((NP, H), jnp.float32),
                   jax.ShapeDtypeStruct((1, 256), jnp.float32)],
    )(dn, nm, m, x)


def _tc_mlp(l1, l2, l3, W1, b1, W2, b2, W3, b3):
    def body(l1_ref, l2_ref, l3_ref, w1_ref, b1_ref, w2_ref, b2_ref,
             w3_ref, b3_ref, o_ref):
        l = l1_ref[...] + l2_ref[...] + l3_ref[...]
        h = jnp.maximum(jnp.dot(l, w1_ref[...], preferred_element_type=jnp.float32)
                        + b1_ref[...], 0.0)
        h = jnp.maximum(jnp.dot(h, w2_ref[...], preferred_element_type=jnp.float32)
                        + b2_ref[...], 0.0)
        o_ref[...] = jnp.dot(h, w3_ref[...], preferred_element_type=jnp.float32) + b3_ref[...]

    return pl.pallas_call(
        body,
        out_shape=jax.ShapeDtypeStruct((1, W3.shape[1]), jnp.float32),
    )(l1, l2, l3, W1, b1.reshape(1, -1), W2, b2.reshape(1, -1), W3, b3.reshape(1, -1))


def kernel(node_feat, node_level, edge_index, edge_feat, W1, b1, W2, b2, W3, b3, W4, b4, gat2_W, gat2_a, gat3_W, gat3_a, gat4_W, gat4_a, Wl1, bl1, Wl2, bl2, Wl3, bl3):
    f32 = jnp.float32
    src = edge_index[0]
    dst = edge_index[1]
    src1d = jnp.concatenate([src, jnp.zeros((EP - E,), jnp.int32)])
    dst1d = jnp.concatenate([dst, jnp.full((EP - E,), DUMP, jnp.int32)])
    src2d = src1d.reshape(EP // 128, 128)
    dst2d = dst1d.reshape(EP // 128, 128)
    efp = jnp.concatenate([edge_feat, jnp.zeros((EP - E, 4), f32)])
    x0 = jnp.concatenate([node_feat.reshape(N, -1), node_level], axis=1)
    x0 = jnp.concatenate([x0, jnp.zeros((NP - N, x0.shape[1]), f32)])
    zeros128 = jnp.zeros((NP, 128), f32)
    zerosN = jnp.zeros((NP,), f32)
    m1 = jnp.concatenate([jnp.ones((N, 1), f32), jnp.zeros((NP - N, 1), f32)])

    def conv(x, m, W, b, linprev, relu_prev, gatW, with_gd=True, deg=None,
             xd_corr=False):
        Fin = x.shape[1]
        Ws, Wd, We = W[:Fin], W[Fin:2 * Fin], W[2 * Fin:]
        xs, xd = _tc_prep(x, m, Ws, Wd, b)
        gs = _sc_gather128(xs, src2d)
        gd = _sc_gather128(xd, dst2d) if with_gd else None
        lin, dstp = _tc_edge(gs, gd, linprev, dst2d, We, relu_prev)
        if deg is None:
            deg = _sc_deg(dstp.reshape(EP), zerosN).T
        acc = _sc_scatter128(lin, dstp, zeros128)
        x_next, h = _tc_fin(acc, deg, xd if xd_corr else None, gatW)
        return x_next, h, lin, dstp, deg

    def gat_topk(h, m, x, ga, k):
        hs, hd, vv = _sc_gat_gather(h.reshape(NP), m.reshape(NP), src1d, dst1d)
        a2 = ga.reshape(1, 2)
        M = _tc_alpha_max(hs.reshape(EP // 128, 128), hd.reshape(EP // 128, 128),
                          vv.reshape(EP // 128, 128), a2)
        a0bc = jnp.broadcast_to(ga[0], (16,))
        a1bc = jnp.broadcast_to(ga[1], (16,))
        den, num = _sc_gat_accum(hs, hd, vv, dst1d, a0bc, a1bc,
                                 M.reshape(128), zerosN)
        return _tc_score_topk(den.T, num.T, m, x, k)

    x1, _, lin1, _, deg1 = conv(x0, None, W1[:44], b1, efp, False, gat2_W)
    x2, h2, lin2, _, _ = conv(x1, None, W2, b2, lin1, True, gat2_W, deg=deg1)
    m2, xm2, l1p = gat_topk(h2, m1, x2, gat2_a, 5000)
    x3, h3, lin3, _, _ = conv(xm2, m2, W3, b3, lin2, True, gat3_W)
    m3, xm3, l2p = gat_topk(h3, m2, x3, gat3_a, 2500)
    x4, h4, _, _, _ = conv(xm3, m3, W4, b4, lin3, True, gat4_W,
                           with_gd=False, xd_corr=True)
    _, _, l3p = gat_topk(h4, m3, x4, gat4_a, 1250)
    return _tc_mlp(l1p, l2p, l3p, Wl1, bl1, Wl2, bl2, Wl3, bl3)
